# Initial kernel scaffold; baseline (speedup 1.0000x reference)
import jax
import jax.numpy as jnp
from jax.experimental import pallas as pl


def kernel(x, y, deg_g, deg_lg, params, edge_index, edge_index_lg, eid2nid):
    raise NotImplementedError("write your pallas kernel here")


# speedup vs baseline: 1.0474x; 1.0474x over previous
import functools

import jax
import jax.numpy as jnp
import numpy as _np
from jax.experimental import pallas as pl

N = 10000
E = 320000
E_LG = 640000
F = 128
TWO_F = 256

_INTERPRET = False
_I0 = _np.int32(0)


def _dense_body(z_ref, z1_ref, z2_ref, agg_ref, deg_ref, w_ref, b_ref,
                t_ref, s1_ref, s2_ref):
    z = z_ref[...]
    xn = (jnp.dot(z, w_ref[0], preferred_element_type=jnp.float32)
          + deg_ref[...] * jnp.dot(z, w_ref[1], preferred_element_type=jnp.float32)
          + jnp.dot(z1_ref[...], w_ref[2], preferred_element_type=jnp.float32)
          + jnp.dot(z2_ref[...], w_ref[3], preferred_element_type=jnp.float32)
          + jnp.dot(agg_ref[...], w_ref[4], preferred_element_type=jnp.float32)
          + b_ref[...])
    t = xn[:, :F] + jax.nn.relu(xn[:, F:])
    t_ref[...] = t
    s1_ref[...] = jnp.sum(t, axis=0, keepdims=True)[None]
    s2_ref[...] = jnp.sum(t * t, axis=0, keepdims=True)[None]


def _dense_stage(z, z1, z2, agg, deg, w_stack, b_sum, block):
    m = z.shape[0]
    grid = m // block
    t, s1, s2 = pl.pallas_call(
        _dense_body,
        grid=(grid,),
        in_specs=[
            pl.BlockSpec((block, F), lambda i: (i, _I0)),
            pl.BlockSpec((block, F), lambda i: (i, _I0)),
            pl.BlockSpec((block, F), lambda i: (i, _I0)),
            pl.BlockSpec((block, F), lambda i: (i, _I0)),
            pl.BlockSpec((block, 1), lambda i: (i, _I0)),
            pl.BlockSpec((5, F, TWO_F), lambda i: (_I0, _I0, _I0)),
            pl.BlockSpec((1, TWO_F), lambda i: (_I0, _I0)),
        ],
        out_specs=[
            pl.BlockSpec((block, F), lambda i: (i, _I0)),
            pl.BlockSpec((1, 1, F), lambda i: (i, _I0, _I0)),
            pl.BlockSpec((1, 1, F), lambda i: (i, _I0, _I0)),
        ],
        out_shape=[
            jax.ShapeDtypeStruct((m, F), jnp.float32),
            jax.ShapeDtypeStruct((grid, 1, F), jnp.float32),
            jax.ShapeDtypeStruct((grid, 1, F), jnp.float32),
        ],
        interpret=_INTERPRET,
    )(z, z1, z2, agg, deg, w_stack, b_sum)
    return t, s1, s2


def _norm_body(t_ref, a_ref, c_ref, o_ref):
    o_ref[...] = t_ref[...] * a_ref[...] + c_ref[...]


def _norm_stage(t, a, c, block):
    m = t.shape[0]
    return pl.pallas_call(
        _norm_body,
        grid=(m // block,),
        in_specs=[
            pl.BlockSpec((block, F), lambda i: (i, _I0)),
            pl.BlockSpec((1, F), lambda i: (_I0, _I0)),
            pl.BlockSpec((1, F), lambda i: (_I0, _I0)),
        ],
        out_specs=pl.BlockSpec((block, F), lambda i: (i, _I0)),
        out_shape=jax.ShapeDtypeStruct((m, F), jnp.float32),
        interpret=_INTERPRET,
    )(t, a, c)


def _branch(z, z1, z2, agg, deg, names, params, bn_w, bn_b, block):
    w_stack = jnp.stack([params[n][0] for n in names])
    b_sum = sum(params[n][1] for n in names).reshape(1, TWO_F).astype(jnp.float32)
    t, s1, s2 = _dense_stage(z, z1, z2, agg, deg, w_stack, b_sum, block)
    m = jnp.float32(z.shape[0])
    mean = jnp.sum(s1, axis=(0, 1)) / m
    var = jnp.sum(s2, axis=(0, 1)) / m - mean * mean
    rstd = jax.lax.rsqrt(var + 1e-5)
    a = (rstd * bn_w).reshape(1, F)
    c = (bn_b - mean * rstd * bn_w).reshape(1, F)
    return _norm_stage(t, a, c, block)


def _segsum(vals, idx, n):
    return jax.ops.segment_sum(vals, idx, num_segments=n)


def kernel(x, y, deg_g, deg_lg, params, edge_index, edge_index_lg, eid2nid):
    src_g = edge_index[0].astype(jnp.int32)
    dst_g = edge_index[1].astype(jnp.int32)
    src_l = edge_index_lg[0].astype(jnp.int32)
    dst_l = edge_index_lg[1].astype(jnp.int32)
    nid = eid2nid.astype(jnp.int32)

    xy = jnp.take(x, nid, axis=0)
    z1g = _segsum(x[src_g], dst_g, N)
    z2g = _segsum(z1g[src_g], dst_g, N)
    yx = _segsum(y, dst_g, N)
    x_out = _branch(x, z1g, z2g, yx, deg_g,
                    ["theta_x", "theta_deg", "theta_0", "theta_1", "theta_y"],
                    params, params["bn_x_w"], params["bn_x_b"], 1000)

    z1l = _segsum(y[src_l], dst_l, E)
    z2l = _segsum(z1l[src_l], dst_l, E)
    xy_agg = _segsum(xy[src_l], dst_l, E)
    y_out = _branch(y, z1l, z2l, xy_agg, deg_lg,
                    ["gamma_y", "gamma_deg", "gamma_0", "gamma_1", "gamma_x"],
                    params, params["bn_y_w"], params["bn_y_b"], 1000)
    return (x_out, y_out)
